# baseline (speedup 1.0000x reference)
import functools

import jax
import jax.numpy as jnp
from jax.experimental import pallas as pl
from jax.experimental.pallas import tpu as pltpu

ALPHA = 0.1
N = 16384
D = 64
BM = 256


def _probe_kernel(adj_ref, inp_ref, h_ref, out_ref):
    i = pl.program_id(0)
    rows = pl.ds(i * BM, BM)
    out_ref[rows, :] = adj_ref[0:BM, 0:D] + ALPHA * h_ref[rows, :]


@functools.partial(jax.jit, static_argnames=())
def kernel(input, adj, h, W):
    del W
    adj_r = adj.reshape(N * 128, 128)
    return pl.pallas_call(
        _probe_kernel,
        grid=(N // BM,),
        in_specs=[
            pl.BlockSpec((BM * 128, 128), lambda i: (i, 0)),
            pl.BlockSpec((N, D), lambda i: (0, 0)),
            pl.BlockSpec((N, D), lambda i: (0, 0)),
        ],
        out_specs=pl.BlockSpec((N, D), lambda i: (0, 0)),
        out_shape=jax.ShapeDtypeStruct((N, D), jnp.float32),
        compiler_params=pltpu.CompilerParams(
            dimension_semantics=("arbitrary",),
        ),
    )(adj_r, input, h)


# speedup vs baseline: 3.6703x; 3.6703x over previous
import functools

import jax
import jax.numpy as jnp
from jax.experimental import pallas as pl
from jax.experimental.pallas import tpu as pltpu

ALPHA = 0.1
N = 16384
D = 64
BM = 256
BK = 4096


def _prop_kernel(adj_ref, inp_ref, h_ref, out_ref):
    i = pl.program_id(0)
    k = pl.program_id(1)
    rows = pl.ds(i * BM, BM)

    @pl.when(k == 0)
    def _init():
        out_ref[rows, :] = ALPHA * h_ref[rows, :]

    out_ref[rows, :] += (1.0 - ALPHA) * jnp.dot(
        adj_ref[...], inp_ref[pl.ds(k * BK, BK), :],
        preferred_element_type=jnp.float32,
    )


@functools.partial(jax.jit, static_argnames=())
def kernel(input, adj, h, W):
    del W
    return pl.pallas_call(
        _prop_kernel,
        grid=(N // BM, N // BK),
        in_specs=[
            pl.BlockSpec((BM, BK), lambda i, k: (i, k)),
            pl.BlockSpec((N, D), lambda i, k: (0, 0)),
            pl.BlockSpec((N, D), lambda i, k: (0, 0)),
        ],
        out_specs=pl.BlockSpec((N, D), lambda i, k: (0, 0)),
        out_shape=jax.ShapeDtypeStruct((N, D), jnp.float32),
        compiler_params=pltpu.CompilerParams(
            dimension_semantics=("arbitrary", "arbitrary"),
        ),
    )(adj, input, h)


# speedup vs baseline: 3.6741x; 1.0010x over previous
import functools

import jax
import jax.numpy as jnp
from jax.experimental import pallas as pl
from jax.experimental.pallas import tpu as pltpu

ALPHA = 0.1
N = 16384
D = 64
BM = 64


def _prop_kernel(adj_ref, inp_ref, h_ref, out_ref):
    out_ref[...] = (1.0 - ALPHA) * jnp.dot(
        adj_ref[...], inp_ref[...], preferred_element_type=jnp.float32
    ) + ALPHA * h_ref[...]


@functools.partial(jax.jit, static_argnames=())
def kernel(input, adj, h, W):
    del W
    return pl.pallas_call(
        _prop_kernel,
        grid=(N // BM,),
        in_specs=[
            pl.BlockSpec((BM, N), lambda i: (i, 0)),
            pl.BlockSpec((N, D), lambda i: (0, 0)),
            pl.BlockSpec((BM, D), lambda i: (i, 0)),
        ],
        out_specs=pl.BlockSpec((BM, D), lambda i: (i, 0)),
        out_shape=jax.ShapeDtypeStruct((N, D), jnp.float32),
        compiler_params=pltpu.CompilerParams(
            dimension_semantics=("arbitrary",),
        ),
    )(adj, input, h)


# speedup vs baseline: 4.4904x; 1.2222x over previous
import functools

import jax
import jax.numpy as jnp
from jax.experimental import pallas as pl
from jax.experimental.pallas import tpu as pltpu

ALPHA = 0.1
N = 16384
D = 64
BM = 128


def _prop_kernel(adj_ref, inp_ref, h_ref, out_ref):
    out_ref[...] = (1.0 - ALPHA) * jnp.dot(
        adj_ref[...], inp_ref[...], preferred_element_type=jnp.float32
    ) + ALPHA * h_ref[...]


@functools.partial(jax.jit, static_argnames=())
def kernel(input, adj, h, W):
    del W
    return pl.pallas_call(
        _prop_kernel,
        grid=(N // BM,),
        in_specs=[
            pl.BlockSpec((BM, N), lambda i: (i, 0)),
            pl.BlockSpec((N, D), lambda i: (0, 0)),
            pl.BlockSpec((BM, D), lambda i: (i, 0)),
        ],
        out_specs=pl.BlockSpec((BM, D), lambda i: (i, 0)),
        out_shape=jax.ShapeDtypeStruct((N, D), jnp.float32),
        compiler_params=pltpu.CompilerParams(
            dimension_semantics=("arbitrary",),
        ),
    )(adj, input, h)
